# baseline (speedup 1.0000x reference)
import functools

import jax
import jax.numpy as jnp
from jax import lax
from jax.experimental import pallas as pl
from jax.experimental.pallas import tpu as pltpu
from jax.experimental.pallas import tpu_sc as plsc

C = 26
V = 100000
VP = 100096
DM = 32
L = 20
NC, NS = 2, 16
NW = NC * NS
LANES = 16

NT = VP // 128
NBLK = C * NT
BPW = (NBLK + NW - 1) // NW

NBC = 16
LW = 5
NWAVE = L // LW
RW = LW * NBC
IC = RW * C
GS = 104
NG = IC // GS


def _format_body(t3_hbm, flat_hbm, inbuf, outbuf, iotas, sem):
    wid = lax.axis_index("s") * NC + lax.axis_index("c")
    lo = wid * BPW
    nb = lax.min(lo + BPW, NBLK) - lo

    def mkiota(s, c2):
        iotas[s, :] = lax.iota(jnp.int32, LANES) * DM + s * (LANES * DM)
        return c2
    lax.fori_loop(0, 8, mkiota, 0)

    def blk(i, carry):
        bk = lo + i
        c = bk // NT
        vt = bk - c * NT
        cps = [
            pltpu.async_copy(
                t3_hbm.at[c, pl.ds(k * 8, 8), pl.ds(vt * 128, 128)],
                inbuf.at[pl.ds(k * 8, 8)],
                sem,
            )
            for k in range(4)
        ]
        for cp in cps:
            cp.wait()

        def trow(d, c2):
            for s in range(8):
                seg = inbuf[d, pl.ds(s * LANES, LANES)]
                plsc.store_scatter(outbuf, [iotas[s, :] + d], seg)
            return c2
        lax.fori_loop(0, DM, trow, 0)

        pltpu.sync_copy(
            outbuf, flat_hbm.at[pl.ds((c * VP + vt * 128) * DM, 128 * DM)])
        return carry

    lax.fori_loop(0, nb, blk, 0)


def _lookup_body(batch, x_hbm, tables_hbm, out_hbm, xv, idxv, rows, outv, sem):
    wid = lax.axis_index("s") * NC + lax.axis_index("c")
    b_per_w = batch // NW
    nchunks = b_per_w // NBC

    def chunk(g, carry):
        b0 = wid * b_per_w + g * NBC
        pltpu.sync_copy(x_hbm.at[:, :, pl.ds(b0, NBC)], xv)

        for w in range(NWAVE):
            def mkidx(t, c2):
                c = t // LW
                dl = t - c * LW
                idxv[pl.ds(t * LANES, LANES)] = xv[c, w * LW + dl, :] + c * VP
                return c2
            lax.fori_loop(0, IC // LANES, mkidx, 0)

            cps = [
                pltpu.async_copy(
                    tables_hbm.at[idxv.at[pl.ds(j * GS, GS)]],
                    rows.at[pl.ds(j * GS, GS)],
                    sem,
                )
                for j in range(NG)
            ]
            for cp in cps:
                cp.wait()

            def srow(q, c2):
                dl = q // NBC
                db = q - dl * NBC
                a0 = rows[q, pl.ds(0, LANES)]
                a1 = rows[q, pl.ds(LANES, LANES)]
                for c in range(1, C):
                    a0 = a0 + rows[q + RW * c, pl.ds(0, LANES)]
                    a1 = a1 + rows[q + RW * c, pl.ds(LANES, LANES)]
                outv[db, w * LW + dl, pl.ds(0, LANES)] = a0
                outv[db, w * LW + dl, pl.ds(LANES, LANES)] = a1
                return c2
            lax.fori_loop(0, RW, srow, 0)

        pltpu.sync_copy(outv, out_hbm.at[pl.ds(b0, NBC)])
        return carry

    lax.fori_loop(0, nchunks, chunk, 0)


@jax.jit
def kernel(x, tables):
    B, sl, c = x.shape
    assert c == C and sl == L and tables.shape == (C, V, DM)
    assert B % (NW * NBC) == 0

    xt = jnp.transpose(x, (2, 1, 0))
    t3 = jnp.transpose(tables, (0, 2, 1))

    mesh = plsc.VectorSubcoreMesh(core_axis_name="c", subcore_axis_name="s")

    fmt = pl.kernel(
        _format_body,
        out_type=jax.ShapeDtypeStruct((C * VP * DM,), jnp.float32),
        mesh=mesh,
        compiler_params=pltpu.CompilerParams(
            use_tc_tiling_on_sc=True, needs_layout_passes=False),
        scratch_types=[
            pltpu.VMEM((DM, 128), jnp.float32),
            pltpu.VMEM((128 * DM,), jnp.float32),
            pltpu.VMEM((8, LANES), jnp.int32),
            pltpu.SemaphoreType.DMA,
        ],
    )
    flat = fmt(t3)
    tables_flat = flat.reshape(C * VP, DM)

    call = pl.kernel(
        functools.partial(_lookup_body, B),
        out_type=jax.ShapeDtypeStruct((B, L, DM), jnp.float32),
        mesh=mesh,
        compiler_params=pltpu.CompilerParams(use_tc_tiling_on_sc=False),
        scratch_types=[
            pltpu.VMEM((C, L, NBC), jnp.int32),
            pltpu.VMEM((IC,), jnp.int32),
            pltpu.VMEM((IC, DM), jnp.float32),
            pltpu.VMEM((NBC, L, DM), jnp.float32),
            pltpu.SemaphoreType.DMA,
        ],
    )
    return call(xt, tables_flat)


# speedup vs baseline: 1.5176x; 1.5176x over previous
import functools

import jax
import jax.numpy as jnp
from jax import lax
from jax.experimental import pallas as pl
from jax.experimental.pallas import tpu as pltpu
from jax.experimental.pallas import tpu_sc as plsc

C = 26
V = 100000
VP = 100096
DM = 32
L = 20
NC, NS = 2, 16
NW = NC * NS
LANES = 16

NT = VP // 128
NBLK = C * NT
BPW = (NBLK + NW - 1) // NW

NBC = 16
LW = 5
NWAVE = L // LW
RW = LW * NBC
IC = RW * C
GS = 104
NG = IC // GS


VB = 256
NSEG = VB // LANES
NT2 = VP // VB
NBLK2 = C * NT2
BPW2 = (NBLK2 + NW - 1) // NW


def _format_body(t3_hbm, flat_hbm, inbuf, outbuf, idxtab, sem_in, sem_out):
    wid = lax.axis_index("s") * NC + lax.axis_index("c")
    lo = wid * BPW2
    nb = lax.min(lo + BPW2, NBLK2) - lo

    def mkiota(s, c2):
        idxtab[s, :] = lax.iota(jnp.int32, LANES) * DM + s * (LANES * DM)
        return c2
    lax.fori_loop(0, NSEG, mkiota, 0)

    def fire_in(bk, p):
        c = bk // NT2
        vt = bk - c * NT2
        for k in range(4):
            pltpu.async_copy(
                t3_hbm.at[c, pl.ds(k * 8, 8), pl.ds(vt * VB, VB)],
                inbuf.at[p, pl.ds(k * 8, 8)],
                sem_in.at[p],
            )

    def drain_in(p):
        for k in range(4):
            pltpu.make_async_copy(
                t3_hbm.at[0, pl.ds(0, 8), pl.ds(0, VB)],
                inbuf.at[p, pl.ds(k * 8, 8)],
                sem_in.at[p],
            ).wait()

    def drain_out():
        pltpu.make_async_copy(
            flat_hbm.at[pl.ds(0, VB * DM)], outbuf, sem_out).wait()

    fire_in(lo, 0)

    def blk(i, carry):
        p = lax.rem(i, 2)
        nxt = lax.min(lo + i + 1, NBLK2 - 1)
        fire_in(nxt, 1 - p)
        drain_in(p)

        @pl.when(i > 0)
        def _():
            drain_out()

        for s in range(NSEG):
            base = idxtab[s, :]
            for d in range(DM):
                seg = inbuf[p, d, pl.ds(s * LANES, LANES)]
                plsc.store_scatter(outbuf, [base + d], seg)

        bk = lo + i
        c = bk // NT2
        vt = bk - c * NT2
        pltpu.async_copy(
            outbuf,
            flat_hbm.at[pl.ds((c * VP + vt * VB) * DM, VB * DM)],
            sem_out,
        )
        return carry

    lax.fori_loop(0, nb, blk, 0)
    drain_in(lax.rem(nb, 2))
    drain_out()


def _lookup_body(batch, x_hbm, tables_hbm, out_hbm, xv, idxv, rows, outv, sem):
    wid = lax.axis_index("s") * NC + lax.axis_index("c")
    b_per_w = batch // NW
    nchunks = b_per_w // NBC

    def chunk(g, carry):
        b0 = wid * b_per_w + g * NBC
        pltpu.sync_copy(x_hbm.at[:, :, pl.ds(b0, NBC)], xv)

        for w in range(NWAVE):
            def mkidx(t, c2):
                c = t // LW
                dl = t - c * LW
                idxv[pl.ds(t * LANES, LANES)] = xv[c, w * LW + dl, :] + c * VP
                return c2
            lax.fori_loop(0, IC // LANES, mkidx, 0)

            cps = [
                pltpu.async_copy(
                    tables_hbm.at[idxv.at[pl.ds(j * GS, GS)]],
                    rows.at[pl.ds(j * GS, GS)],
                    sem,
                )
                for j in range(NG)
            ]
            for cp in cps:
                cp.wait()

            def srow(q, c2):
                dl = q // NBC
                db = q - dl * NBC
                a0 = rows[q, pl.ds(0, LANES)]
                a1 = rows[q, pl.ds(LANES, LANES)]
                for c in range(1, C):
                    a0 = a0 + rows[q + RW * c, pl.ds(0, LANES)]
                    a1 = a1 + rows[q + RW * c, pl.ds(LANES, LANES)]
                outv[db, w * LW + dl, pl.ds(0, LANES)] = a0
                outv[db, w * LW + dl, pl.ds(LANES, LANES)] = a1
                return c2
            lax.fori_loop(0, RW, srow, 0)

        pltpu.sync_copy(outv, out_hbm.at[pl.ds(b0, NBC)])
        return carry

    lax.fori_loop(0, nchunks, chunk, 0)


@jax.jit
def kernel(x, tables):
    B, sl, c = x.shape
    assert c == C and sl == L and tables.shape == (C, V, DM)
    assert B % (NW * NBC) == 0

    xt = jnp.transpose(x, (2, 1, 0))
    t3 = jnp.transpose(tables, (0, 2, 1))

    mesh = plsc.VectorSubcoreMesh(core_axis_name="c", subcore_axis_name="s")

    fmt = pl.kernel(
        _format_body,
        out_type=jax.ShapeDtypeStruct((C * VP * DM,), jnp.float32),
        mesh=mesh,
        compiler_params=pltpu.CompilerParams(
            use_tc_tiling_on_sc=True, needs_layout_passes=False),
        scratch_types=[
            pltpu.VMEM((2, DM, VB), jnp.float32),
            pltpu.VMEM((VB * DM,), jnp.float32),
            pltpu.VMEM((NSEG, LANES), jnp.int32),
            pltpu.SemaphoreType.DMA((2,)),
            pltpu.SemaphoreType.DMA,
        ],
    )
    flat = fmt(t3)
    tables_flat = flat.reshape(C * VP, DM)

    call = pl.kernel(
        functools.partial(_lookup_body, B),
        out_type=jax.ShapeDtypeStruct((B, L, DM), jnp.float32),
        mesh=mesh,
        compiler_params=pltpu.CompilerParams(use_tc_tiling_on_sc=False),
        scratch_types=[
            pltpu.VMEM((C, L, NBC), jnp.int32),
            pltpu.VMEM((IC,), jnp.int32),
            pltpu.VMEM((IC, DM), jnp.float32),
            pltpu.VMEM((NBC, L, DM), jnp.float32),
            pltpu.SemaphoreType.DMA,
        ],
    )
    return call(xt, tables_flat)
